# baseline (speedup 1.0000x reference)
import functools

import jax
import jax.numpy as jnp
from jax import lax
from jax.experimental import pallas as pl
from jax.experimental.pallas import tpu as pltpu
from jax.experimental.pallas import tpu_sc as plsc

B, K, CH = 64, 32768, 16
L = 16
NC, NS = 2, 16
NW = NC * NS
NB_SC = 24
NBLK_SC = NB_SC * 2
BLK_PER_W = 2
NWA = NBLK_SC // BLK_PER_W
KT = 256
TK = 32
NCHUNK = KT // TK
CHUNKS_PER_W = BLK_PER_W * NCHUNK
IMAX = 2**31 - 1


def _scan_chunk(buf, t0, carries):
    iota = lax.iota(jnp.int32, L)

    def tbody(t, flat):
        bests = list(flat[0::2])
        bidxs = list(flat[1::2])
        kbase = (t0 + t) * 128
        for j in range(8):
            idx = iota + (kbase + j * L)
            for c in range(8):
                v = buf[t, c, pl.ds(j * L, L)]
                mask = v < bests[c]
                bests[c] = jnp.minimum(bests[c], v)
                bidxs[c] = jnp.where(mask, idx, bidxs[c])
        return tuple(y for b, i in zip(bests, bidxs) for y in (b, i))

    flat = lax.fori_loop(
        0, TK, tbody, tuple(y for carry in carries for y in carry)
    )
    return [(flat[2 * c], flat[2 * c + 1]) for c in range(8)]


@functools.partial(
    pl.kernel,
    out_type=jax.ShapeDtypeStruct((NB_SC, 1, CH), jnp.int32),
    mesh=plsc.VectorSubcoreMesh(core_axis_name="c", subcore_axis_name="s"),
    scratch_types=[
        pltpu.VMEM((TK, 8, 128), jnp.float32),
        pltpu.VMEM((TK, 8, 128), jnp.float32),
        pltpu.VMEM((BLK_PER_W // 2, 1, CH), jnp.int32),
        pltpu.SemaphoreType.DMA,
        pltpu.SemaphoreType.DMA,
    ],
    compiler_params=pltpu.CompilerParams(
        use_tc_tiling_on_sc=False, needs_layout_passes=False
    ),
)
def _argmin_sc(z_hbm, out_hbm, buf0, buf1, outbuf, sem0, sem1):
    wid = lax.axis_index("s") * NC + lax.axis_index("c")

    @pl.when(wid < NWA)
    def _body():
        _worker(z_hbm, out_hbm, buf0, buf1, outbuf, sem0, sem1, wid)


def _worker(z_hbm, out_hbm, buf0, buf1, outbuf, sem0, sem1, wid):
    blk0 = wid * BLK_PER_W

    bufs = (buf0, buf1)
    sems = (sem0, sem1)

    def start(g, parity):
        gc = jnp.minimum(g, CHUNKS_PER_W - 1)
        blk = blk0 + gc // NCHUNK
        t0 = (gc % NCHUNK) * TK
        pltpu.async_copy(z_hbm.at[blk, pl.ds(t0, TK)], bufs[parity],
                         sems[parity])

    def wait(parity):
        pltpu.make_async_copy(
            z_hbm.at[0, pl.ds(0, TK)], bufs[parity], sems[parity]
        ).wait()

    start(0, 0)
    start(1, 1)

    lane = lax.iota(jnp.int32, L)

    def blk_body(blk, acc):
        carries = [(jnp.full((L,), jnp.inf, jnp.float32),
                    jnp.full((L,), 0, jnp.int32)) for _ in range(8)]

        def pbody(p, flat):
            carries = [(flat[2 * c], flat[2 * c + 1]) for c in range(8)]
            for parity in range(2):
                g = blk * NCHUNK + 2 * p + parity
                wait(parity)
                carries = _scan_chunk(bufs[parity], (2 * p + parity) * TK,
                                      carries)
                start(g + 2, parity)
            return tuple(y for carry in carries for y in carry)

        flat = lax.fori_loop(0, NCHUNK // 2, pbody,
                             tuple(y for carry in carries for y in carry))

        c_grp = lax.rem(blk, 2)
        for c in range(8):
            best, bidx = flat[2 * c], flat[2 * c + 1]
            m = lax.reduce_min(best, (0,))
            cand = jnp.where(best == m, bidx, IMAX)
            r = lax.reduce_min(cand, (0,))
            acc = jnp.where(lane == 8 * c_grp + c, r, acc)

        @pl.when(c_grp == 1)
        def _store():
            outbuf[lax.div(blk, 2), 0] = acc

        return acc

    lax.fori_loop(0, BLK_PER_W, blk_body, jnp.full((L,), 0, jnp.int32))

    wait(0)
    wait(1)
    pltpu.sync_copy(outbuf, out_hbm.at[pl.ds(wid * (BLK_PER_W // 2),
                                             BLK_PER_W // 2)])


def _tc_body(z_ref, o_ref):
    for g in range(8):
        v = z_ref[g]
        m1 = jnp.min(v, axis=0)
        i0 = lax.broadcasted_iota(jnp.int32, (KT, 8, 128), 0)
        t_first = jnp.min(jnp.where(v == m1[None], i0, IMAX), axis=0)
        m = jnp.min(m1, axis=1, keepdims=True)
        kin = lax.broadcasted_iota(jnp.int32, (8, 128), 1)
        cand = jnp.where(m1 == m, t_first * 128 + kin, IMAX)
        o_ref[g // 2, 0, g % 2] = jnp.min(cand, axis=1)


_argmin_tc = pl.pallas_call(
    _tc_body,
    grid=((B - NB_SC) // 4,),
    in_specs=[
        pl.BlockSpec((8, KT, 8, 128), lambda i: (NB_SC // 4 + i, 0, 0, 0))
    ],
    out_specs=pl.BlockSpec((4, 1, 2, 8), lambda i: (i, 0, 0, 0)),
    out_shape=jax.ShapeDtypeStruct((B - NB_SC, 1, 2, 8), jnp.int32),
)


def kernel(x):
    z = x.reshape(B, KT, 128, 2, 8).transpose(0, 3, 1, 4, 2)
    z = z.reshape(2 * B, KT, 8, 128)
    lo = _argmin_sc(z)
    hi = _argmin_tc(z).reshape(B - NB_SC, 1, CH)
    return jnp.concatenate([lo, hi], axis=0).astype(jnp.int64)


# speedup vs baseline: 1.0076x; 1.0076x over previous
import functools

import jax
import jax.numpy as jnp
from jax import lax
from jax.experimental import pallas as pl
from jax.experimental.pallas import tpu as pltpu
from jax.experimental.pallas import tpu_sc as plsc

B, K, CH = 64, 32768, 16
L = 16
NC, NS = 2, 16
NW = NC * NS
NB_SC = 20
NBLK_SC = NB_SC * 2
BLK_PER_W = 2
NWA = NBLK_SC // BLK_PER_W
KT = 256
TK = 32
NCHUNK = KT // TK
CHUNKS_PER_W = BLK_PER_W * NCHUNK
IMAX = 2**31 - 1


def _scan_chunk(buf, t0, carries):
    iota = lax.iota(jnp.int32, L)

    def tbody(t, flat):
        bests = list(flat[0::2])
        bidxs = list(flat[1::2])
        kbase = (t0 + t) * 128
        for j in range(8):
            idx = iota + (kbase + j * L)
            for c in range(8):
                v = buf[t, c, pl.ds(j * L, L)]
                mask = v < bests[c]
                bests[c] = jnp.minimum(bests[c], v)
                bidxs[c] = jnp.where(mask, idx, bidxs[c])
        return tuple(y for b, i in zip(bests, bidxs) for y in (b, i))

    flat = lax.fori_loop(
        0, TK, tbody, tuple(y for carry in carries for y in carry)
    )
    return [(flat[2 * c], flat[2 * c + 1]) for c in range(8)]


@functools.partial(
    pl.kernel,
    out_type=jax.ShapeDtypeStruct((NB_SC, 1, CH), jnp.int32),
    mesh=plsc.VectorSubcoreMesh(core_axis_name="c", subcore_axis_name="s"),
    scratch_types=[
        pltpu.VMEM((TK, 8, 128), jnp.float32),
        pltpu.VMEM((TK, 8, 128), jnp.float32),
        pltpu.VMEM((BLK_PER_W // 2, 1, CH), jnp.int32),
        pltpu.SemaphoreType.DMA,
        pltpu.SemaphoreType.DMA,
    ],
    compiler_params=pltpu.CompilerParams(
        use_tc_tiling_on_sc=False, needs_layout_passes=False
    ),
)
def _argmin_sc(z_hbm, out_hbm, buf0, buf1, outbuf, sem0, sem1):
    wid = lax.axis_index("s") * NC + lax.axis_index("c")

    @pl.when(wid < NWA)
    def _body():
        _worker(z_hbm, out_hbm, buf0, buf1, outbuf, sem0, sem1, wid)


def _worker(z_hbm, out_hbm, buf0, buf1, outbuf, sem0, sem1, wid):
    blk0 = wid * BLK_PER_W

    bufs = (buf0, buf1)
    sems = (sem0, sem1)

    def start(g, parity):
        gc = jnp.minimum(g, CHUNKS_PER_W - 1)
        blk = blk0 + gc // NCHUNK
        t0 = (gc % NCHUNK) * TK
        pltpu.async_copy(z_hbm.at[blk, pl.ds(t0, TK)], bufs[parity],
                         sems[parity])

    def wait(parity):
        pltpu.make_async_copy(
            z_hbm.at[0, pl.ds(0, TK)], bufs[parity], sems[parity]
        ).wait()

    start(0, 0)
    start(1, 1)

    lane = lax.iota(jnp.int32, L)

    def blk_body(blk, acc):
        carries = [(jnp.full((L,), jnp.inf, jnp.float32),
                    jnp.full((L,), 0, jnp.int32)) for _ in range(8)]

        def pbody(p, flat):
            carries = [(flat[2 * c], flat[2 * c + 1]) for c in range(8)]
            for parity in range(2):
                g = blk * NCHUNK + 2 * p + parity
                wait(parity)
                carries = _scan_chunk(bufs[parity], (2 * p + parity) * TK,
                                      carries)
                start(g + 2, parity)
            return tuple(y for carry in carries for y in carry)

        flat = lax.fori_loop(0, NCHUNK // 2, pbody,
                             tuple(y for carry in carries for y in carry))

        c_grp = lax.rem(blk, 2)
        for c in range(8):
            best, bidx = flat[2 * c], flat[2 * c + 1]
            m = lax.reduce_min(best, (0,))
            cand = jnp.where(best == m, bidx, IMAX)
            r = lax.reduce_min(cand, (0,))
            acc = jnp.where(lane == 8 * c_grp + c, r, acc)

        @pl.when(c_grp == 1)
        def _store():
            outbuf[lax.div(blk, 2), 0] = acc

        return acc

    lax.fori_loop(0, BLK_PER_W, blk_body, jnp.full((L,), 0, jnp.int32))

    wait(0)
    wait(1)
    pltpu.sync_copy(outbuf, out_hbm.at[pl.ds(wid * (BLK_PER_W // 2),
                                             BLK_PER_W // 2)])


def _tc_body(z_ref, o_ref):
    for g in range(8):
        v = z_ref[g]
        m1 = jnp.min(v, axis=0)
        i0 = lax.broadcasted_iota(jnp.int32, (KT, 8, 128), 0)
        t_first = jnp.min(jnp.where(v == m1[None], i0, IMAX), axis=0)
        m = jnp.min(m1, axis=1, keepdims=True)
        kin = lax.broadcasted_iota(jnp.int32, (8, 128), 1)
        cand = jnp.where(m1 == m, t_first * 128 + kin, IMAX)
        o_ref[g // 2, 0, g % 2] = jnp.min(cand, axis=1)


_argmin_tc = pl.pallas_call(
    _tc_body,
    grid=((B - NB_SC) // 4,),
    in_specs=[
        pl.BlockSpec((8, KT, 8, 128), lambda i: (NB_SC // 4 + i, 0, 0, 0))
    ],
    out_specs=pl.BlockSpec((4, 1, 2, 8), lambda i: (i, 0, 0, 0)),
    out_shape=jax.ShapeDtypeStruct((B - NB_SC, 1, 2, 8), jnp.int32),
)


def kernel(x):
    z = x.reshape(B, KT, 128, 2, 8).transpose(0, 3, 1, 4, 2)
    z = z.reshape(2 * B, KT, 8, 128)
    lo = _argmin_sc(z)
    hi = _argmin_tc(z).reshape(B - NB_SC, 1, CH)
    return jnp.concatenate([lo, hi], axis=0).astype(jnp.int64)
